# baseline (speedup 1.0000x reference)
import functools

import jax
import jax.numpy as jnp
from jax import lax
from jax.experimental import pallas as pl
from jax.experimental.pallas import tpu as pltpu
from jax.experimental.pallas import tpu_sc as plsc

_GS = 4


def _vq_block(z_ref, e_ref, en_ref, w_ref, g4_ref, idx_ref):
    zb = z_ref[...]
    e = e_ref[...]
    en = en_ref[...]
    w = w_ref[...]
    g4 = g4_ref[...]
    zn = jnp.sum(zb * zb, axis=-1, keepdims=True)
    dots = jnp.dot(zb, e.T, preferred_element_type=jnp.float32)
    d = zn + en[None, :] - 2.0 * dots
    m = jnp.min(d, axis=-1, keepdims=True)
    mask = (d == m).astype(jnp.float32)
    bits = jnp.dot(mask, w, preferred_element_type=jnp.float32)
    ebias = lax.bitcast_convert_type(bits, jnp.int32) >> 23
    kcand = g4[None, :] + (127 - ebias)
    kcf = jnp.where(bits > 0.0, kcand.astype(jnp.float32), float(d.shape[1]))
    idxf = jnp.min(kcf, axis=-1)
    idx_ref[...] = idxf.astype(jnp.int32)


def _argmin_indices(zf, emb_weight):
    N, D = zf.shape
    K = emb_weight.shape[0]
    G = K // _GS
    en = jnp.sum(emb_weight ** 2, axis=-1)
    karange = jnp.arange(K)
    w = jnp.where(
        (karange // _GS)[:, None] == jnp.arange(G)[None, :],
        jnp.exp2(-(karange % _GS).astype(jnp.float32))[:, None],
        0.0,
    )
    g4 = jnp.arange(G, dtype=jnp.int32) * _GS
    TB = 1024
    grid = N // TB

    return pl.pallas_call(
        _vq_block,
        grid=(grid,),
        in_specs=[
            pl.BlockSpec((TB, D), lambda i: (i, 0)),
            pl.BlockSpec((K, D), lambda i: (0, 0)),
            pl.BlockSpec((K,), lambda i: (0,)),
            pl.BlockSpec((K, G), lambda i: (0, 0)),
            pl.BlockSpec((G,), lambda i: (0,)),
        ],
        out_specs=pl.BlockSpec((TB,), lambda i: (i,)),
        out_shape=jax.ShapeDtypeStruct((N,), jnp.int32),
    )(zf, emb_weight, en, w, g4)


def _sc_gather(table, idx):
    B = idx.shape[0]
    K, D = table.shape
    info = plsc.get_sparse_core_info()
    nw = info.num_cores * info.num_subcores
    b_per_w = B // nw
    mesh = plsc.VectorSubcoreMesh(
        core_axis_name="c", subcore_axis_name="s", num_cores=info.num_cores
    )

    @functools.partial(
        pl.kernel,
        mesh=mesh,
        compiler_params=pltpu.CompilerParams(use_tc_tiling_on_sc=False),
        out_type=jax.ShapeDtypeStruct((B, D), jnp.float32),
        scratch_types=[
            pltpu.VMEM((b_per_w,), jnp.int32),
            pltpu.VMEM((b_per_w, D), jnp.float32),
            pltpu.SemaphoreType.DMA,
        ],
    )
    def gather_kernel(table_hbm, idx_hbm, out_hbm, idx_v, rows_v, sem):
        wid = lax.axis_index("s") * info.num_cores + lax.axis_index("c")
        base = wid * b_per_w
        pltpu.sync_copy(idx_hbm.at[pl.ds(base, b_per_w)], idx_v)
        pltpu.async_copy(table_hbm.at[idx_v], rows_v, sem).wait()
        pltpu.sync_copy(rows_v, out_hbm.at[pl.ds(base, b_per_w)])

    return gather_kernel(table, idx)


def kernel(z, emb_weight):
    B, T, D = z.shape
    zf = z.reshape(B * T, D)
    idx = _argmin_indices(zf, emb_weight)
    zq = _sc_gather(emb_weight, idx)
    return zq.reshape(B, T, D), idx.reshape(B, T)


# speedup vs baseline: 1.6585x; 1.6585x over previous
import jax
import jax.numpy as jnp
from jax.experimental import pallas as pl


def _vq_block(z_ref, e_ref, en_ref, kf_ref, zq_ref, idx_ref):
    zb = z_ref[...]
    e = e_ref[...]
    en = en_ref[...]
    kf = kf_ref[...]
    zn = jnp.sum(zb * zb, axis=-1, keepdims=True)
    dots = jnp.dot(zb, e.T, preferred_element_type=jnp.float32)
    d = zn + en[None, :] - 2.0 * dots
    m = jnp.min(d, axis=-1, keepdims=True)
    kiof = kf[None, :]
    idxf = jnp.min(jnp.where(d == m, kiof, float(d.shape[1])), axis=-1)
    idx_ref[...] = idxf.astype(jnp.int32)
    oh = (kiof == idxf[:, None]).astype(jnp.float32)
    zq = jnp.dot(oh, e, preferred_element_type=jnp.float32)
    zq_ref[...] = zb + (zq - zb)


def kernel(z, emb_weight):
    B, T, D = z.shape
    K = emb_weight.shape[0]
    zf = z.reshape(B * T, D)
    en = jnp.sum(emb_weight ** 2, axis=-1)
    kf = jnp.arange(K, dtype=jnp.float32)
    TB = 1024
    grid = (B * T) // TB

    zq, idx = pl.pallas_call(
        _vq_block,
        grid=(grid,),
        in_specs=[
            pl.BlockSpec((TB, D), lambda i: (i, 0)),
            pl.BlockSpec((K, D), lambda i: (0, 0)),
            pl.BlockSpec((K,), lambda i: (0,)),
            pl.BlockSpec((K,), lambda i: (0,)),
        ],
        out_specs=[
            pl.BlockSpec((TB, D), lambda i: (i, 0)),
            pl.BlockSpec((TB,), lambda i: (i,)),
        ],
        out_shape=[
            jax.ShapeDtypeStruct((B * T, D), jnp.float32),
            jax.ShapeDtypeStruct((B * T,), jnp.int32),
        ],
    )(zf, emb_weight, en, kf)
    return zq.reshape(B, T, D), idx.reshape(B, T)
